# Initial kernel scaffold; baseline (speedup 1.0000x reference)
import jax
import jax.numpy as jnp
from jax.experimental import pallas as pl


def kernel(x, edge_index, batch, W1, b1, W2, b2, fc1_W, fc1_b, fc2_W, fc2_b):
    raise NotImplementedError("write your pallas kernel here")


# speedup vs baseline: 9.1473x; 9.1473x over previous
import functools
import jax
import jax.numpy as jnp
from jax import lax
from jax.experimental import pallas as pl
from jax.experimental.pallas import tpu as pltpu
from jax.experimental.pallas import tpu_sc as plsc

N_NODES = 10000
N_GRAPHS = 64
D = 128
NP = 10240
NCORE = 2
NSUB = 16
NW = NCORE * NSUB
CHUNK = 128
CPW = 80
E_PAD = NW * CPW * CHUNK
ROWS_PT = NP // NSUB
WV = 8
NWAVE = CPW // WV

_f32 = jnp.float32


def _deg_body(dstr_hbm, z1d_hbm, deg_hbm, dst_v, ones_v, sem, acc):
    c = lax.axis_index("c")
    s = lax.axis_index("s")
    w = c * NSUB + s
    pltpu.sync_copy(dstr_hbm.at[pl.ds(w * CPW, CPW)], dst_v)
    rows = pl.ds(s * ROWS_PT, ROWS_PT)
    pltpu.sync_copy(z1d_hbm.at[rows], acc.at[rows])
    for i in range(CHUNK // 16):
        ones_v[pl.ds(i * 16, 16)] = jnp.ones((16,), _f32)
    plsc.subcore_barrier()

    def body(i, carry):
        for b in range(4):
            pltpu.async_copy(ones_v, acc.at[dst_v.at[i * 4 + b]], sem, add=True)
        for b in range(4):
            pltpu.make_async_copy(ones_v, acc.at[pl.ds(0, CHUNK)], sem).wait()
        return carry

    lax.fori_loop(0, CPW // 4, body, 0)
    plsc.subcore_barrier()
    pltpu.sync_copy(acc.at[rows], deg_hbm.at[c].at[rows])


def _mp_body(y_hbm, srcr_hbm, dstr_hbm, out_hbm,
             sidx, didx, bufs, gsem, ssem, isem, acc):
    c = lax.axis_index("c")
    s = lax.axis_index("s")
    w = c * NSUB + s
    base = w * CPW
    pltpu.sync_copy(srcr_hbm.at[pl.ds(base, CPW)], sidx)
    rows = pl.ds(s * ROWS_PT, ROWS_PT)

    pltpu.sync_copy(y_hbm.at[rows], acc.at[rows])
    pltpu.async_copy(dstr_hbm.at[pl.ds(base, WV)], didx.at[0], isem)
    plsc.subcore_barrier()

    def wait_gather(b):
        pltpu.make_async_copy(y_hbm.at[pl.ds(0, CHUNK)], bufs.at[b], gsem.at[b]).wait()

    def wait_scatter(b):
        pltpu.make_async_copy(bufs.at[b], acc.at[pl.ds(0, CHUNK)], ssem.at[b]).wait()

    def wait_idx(m):
        pltpu.make_async_copy(dstr_hbm.at[pl.ds(0, WV)], didx.at[m], isem).wait()

    for b in range(2):
        pltpu.async_copy(y_hbm.at[sidx.at[b]], bufs.at[b], gsem.at[b])

    def wave(wv, carry):
        m = lax.rem(wv, 2)
        wait_idx(m)

        @pl.when(wv < NWAVE - 1)
        def _():
            pltpu.async_copy(dstr_hbm.at[pl.ds(base + (wv + 1) * WV, WV)],
                             didx.at[lax.rem(wv + 1, 2)], isem)

        for k in range(WV):
            b = k % 2
            j = wv * WV + k
            wait_gather(b)
            pltpu.async_copy(bufs.at[b], acc.at[didx.at[m].at[k]],
                             ssem.at[b], add=True)
            wait_scatter(b)

            @pl.when(j + 2 < CPW)
            def _():
                pltpu.async_copy(y_hbm.at[sidx.at[j + 2]], bufs.at[b], gsem.at[b])
        return carry

    lax.fori_loop(0, NWAVE, wave, 0)
    plsc.subcore_barrier()
    pltpu.sync_copy(acc.at[rows], out_hbm.at[c].at[rows])


_sc_mesh = plsc.VectorSubcoreMesh(core_axis_name="c", subcore_axis_name="s")

_deg_call = pl.kernel(
    _deg_body,
    out_type=jax.ShapeDtypeStruct((NCORE, NP), _f32),
    mesh=_sc_mesh,
    scratch_types=[
        pltpu.VMEM((CPW, CHUNK), jnp.int32),
        pltpu.VMEM((CHUNK,), _f32),
        pltpu.SemaphoreType.DMA,
        pltpu.VMEM_SHARED((NP,), _f32),
    ],
)

_mp_call = pl.kernel(
    _mp_body,
    out_type=jax.ShapeDtypeStruct((NCORE, NP, D), _f32),
    mesh=_sc_mesh,
    scratch_types=[
        pltpu.VMEM((CPW, CHUNK), jnp.int32),
        pltpu.VMEM((2, WV, CHUNK), jnp.int32),
        pltpu.VMEM((2, CHUNK, D), _f32),
        pltpu.SemaphoreType.DMA((2,)),
        pltpu.SemaphoreType.DMA((2,)),
        pltpu.SemaphoreType.DMA,
        pltpu.VMEM_SHARED((NP, D), _f32),
    ],
)


_BLK = 1024
_GRID = NP // _BLK


def _tc1_body(x_ref, w_ref, d0_ref, d1_ref, m_ref, y_ref, dinv_ref):
    deg = d0_ref[...] + d1_ref[...] + 1.0
    dinv = lax.rsqrt(deg) * m_ref[...]
    dinv_ref[...] = dinv
    xw = jnp.dot(x_ref[...], w_ref[...], preferred_element_type=_f32,
                 precision=lax.Precision.HIGHEST)
    y_ref[...] = xw * dinv[:, None]


def _tc2_body(a0_ref, a1_ref, y1_ref, dinv_ref, b_ref, w_ref, y_ref):
    dinv = dinv_ref[...]
    acc = a0_ref[...] + a1_ref[...] - y1_ref[...]
    h = jnp.maximum(dinv[:, None] * acc + b_ref[...][None, :], 0.0)
    y_ref[...] = jnp.dot(h, w_ref[...], preferred_element_type=_f32,
                         precision=lax.Precision.HIGHEST) * dinv[:, None]


def _tc3_body(a0_ref, a1_ref, y2_ref, dinv_ref, b_ref, batch_ref, w1_ref, b1_ref,
              w2_ref, b2_ref, out_ref):
    dinv = dinv_ref[...]
    acc = a0_ref[...] + a1_ref[...] - y2_ref[...]
    h = jnp.maximum(dinv[:, None] * acc + b_ref[...][None, :], 0.0)
    gid = lax.broadcasted_iota(jnp.int32, (N_GRAPHS, NP), 0)
    onehot = (gid == batch_ref[...][None, :]).astype(_f32)
    sums = jnp.dot(onehot, h, preferred_element_type=_f32,
                   precision=lax.Precision.HIGHEST)
    counts = jnp.sum(onehot, axis=1)
    g = sums / jnp.maximum(counts, 1.0)[:, None]
    g = jnp.maximum(jnp.dot(g, w1_ref[...], preferred_element_type=_f32,
                            precision=lax.Precision.HIGHEST) + b1_ref[...][None, :], 0.0)
    out_ref[...] = jnp.dot(g, w2_ref[...], preferred_element_type=_f32,
                           precision=lax.Precision.HIGHEST) + b2_ref[...][None, :]


_row_spec = pl.BlockSpec((_BLK, D), lambda i: (i, 0))
_vec_spec = pl.BlockSpec((_BLK,), lambda i: (i,))
_full_mat = pl.BlockSpec((D, D), lambda i: (0, 0))
_full_vec = pl.BlockSpec((D,), lambda i: (0,))

_tc1_call = pl.pallas_call(
    _tc1_body,
    grid=(_GRID,),
    in_specs=[_row_spec, _full_mat, _vec_spec, _vec_spec, _vec_spec],
    out_specs=(_row_spec, _vec_spec),
    out_shape=(jax.ShapeDtypeStruct((NP, D), _f32),
               jax.ShapeDtypeStruct((NP,), _f32)),
)

_tc2_call = pl.pallas_call(
    _tc2_body,
    grid=(_GRID,),
    in_specs=[_row_spec, _row_spec, _row_spec, _vec_spec, _full_vec, _full_mat],
    out_specs=_row_spec,
    out_shape=jax.ShapeDtypeStruct((NP, D), _f32),
)

_tc3_call = pl.pallas_call(
    _tc3_body,
    out_shape=jax.ShapeDtypeStruct((N_GRAPHS, 10), _f32),
)


def kernel(x, edge_index, batch, W1, b1, W2, b2, fc1_W, fc1_b, fc2_W, fc2_b):
    src = edge_index[0].astype(jnp.int32)
    dst = edge_index[1].astype(jnp.int32)
    pad_e = E_PAD - src.shape[0]
    pad_idx = jnp.full((pad_e,), N_NODES, jnp.int32)
    src_r = jnp.concatenate([src, pad_idx]).reshape(NW * CPW, CHUNK)
    dst_r = jnp.concatenate([dst, pad_idx]).reshape(NW * CPW, CHUNK)

    x_p = jnp.pad(x, ((0, NP - N_NODES), (0, 0)))
    batch_p = jnp.pad(batch.astype(jnp.int32), (0, NP - N_NODES),
                      constant_values=N_GRAPHS)
    mask = (jnp.arange(NP, dtype=jnp.int32) < N_NODES).astype(_f32)
    z1d = jnp.zeros((NP,), _f32)

    deg = _deg_call(dst_r, z1d)
    y1, dinv = _tc1_call(x_p, W1, deg[0], deg[1], mask)
    a = _mp_call(y1, src_r, dst_r)
    y2 = _tc2_call(a[0], a[1], y1, dinv, b1, W2)
    a = _mp_call(y2, src_r, dst_r)
    return _tc3_call(a[0], a[1], y2, dinv, b2, batch_p, fc1_W, fc1_b, fc2_W, fc2_b)


# speedup vs baseline: 9.6279x; 1.0525x over previous
import functools
import jax
import jax.numpy as jnp
from jax import lax
from jax.experimental import pallas as pl
from jax.experimental.pallas import tpu as pltpu
from jax.experimental.pallas import tpu_sc as plsc

N_NODES = 10000
N_GRAPHS = 64
D = 128
NP = 10240
NCORE = 2
NSUB = 16
NW = NCORE * NSUB
CHUNK = 128
CPW = 80
E_PAD = NW * CPW * CHUNK
ROWS_PT = NP // NSUB
WV = 8
NWAVE = CPW // WV

_f32 = jnp.float32


def _deg_body(dstr_hbm, z1d_hbm, deg_hbm, dst_v, ones_v, sem, acc):
    c = lax.axis_index("c")
    s = lax.axis_index("s")
    w = c * NSUB + s
    pltpu.sync_copy(dstr_hbm.at[pl.ds(w * CPW, CPW)], dst_v)
    rows = pl.ds(s * ROWS_PT, ROWS_PT)
    pltpu.sync_copy(z1d_hbm.at[rows], acc.at[rows])
    for i in range(CHUNK // 16):
        ones_v[pl.ds(i * 16, 16)] = jnp.ones((16,), _f32)
    plsc.subcore_barrier()

    def body(i, carry):
        for b in range(4):
            pltpu.async_copy(ones_v, acc.at[dst_v.at[i * 4 + b]], sem, add=True)
        for b in range(4):
            pltpu.make_async_copy(ones_v, acc.at[pl.ds(0, CHUNK)], sem).wait()
        return carry

    lax.fori_loop(0, CPW // 4, body, 0)
    plsc.subcore_barrier()
    pltpu.sync_copy(acc.at[rows], deg_hbm.at[c].at[rows])


def _mp_body(y_hbm, srcr_hbm, dstr_hbm, out_hbm,
             sidx, didx, bufs, gsem, ssem, isem, acc):
    c = lax.axis_index("c")
    s = lax.axis_index("s")
    w = c * NSUB + s
    base = w * CPW
    pltpu.sync_copy(srcr_hbm.at[pl.ds(base, CPW)], sidx)
    rows = pl.ds(s * ROWS_PT, ROWS_PT)

    pltpu.sync_copy(y_hbm.at[rows], acc.at[rows])
    pltpu.async_copy(dstr_hbm.at[pl.ds(base, WV)], didx.at[0], isem)
    plsc.subcore_barrier()

    def wait_gather(b):
        pltpu.make_async_copy(y_hbm.at[pl.ds(0, CHUNK)], bufs.at[b], gsem.at[b]).wait()

    def wait_scatter(b):
        pltpu.make_async_copy(bufs.at[b], acc.at[pl.ds(0, CHUNK)], ssem.at[b]).wait()

    def wait_idx(m):
        pltpu.make_async_copy(dstr_hbm.at[pl.ds(0, WV)], didx.at[m], isem).wait()

    for b in range(2):
        pltpu.async_copy(y_hbm.at[sidx.at[b]], bufs.at[b], gsem.at[b])

    def wave(wv, carry):
        m = lax.rem(wv, 2)
        wait_idx(m)

        @pl.when(wv < NWAVE - 1)
        def _():
            pltpu.async_copy(dstr_hbm.at[pl.ds(base + (wv + 1) * WV, WV)],
                             didx.at[lax.rem(wv + 1, 2)], isem)

        for k in range(WV):
            b = k % 2
            j = wv * WV + k
            wait_gather(b)
            pltpu.async_copy(bufs.at[b], acc.at[didx.at[m].at[k]],
                             ssem.at[b], add=True)
            wait_scatter(b)

            @pl.when(j + 2 < CPW)
            def _():
                pltpu.async_copy(y_hbm.at[sidx.at[j + 2]], bufs.at[b], gsem.at[b])
        return carry

    lax.fori_loop(0, NWAVE, wave, 0)
    plsc.subcore_barrier()
    pltpu.sync_copy(acc.at[rows], out_hbm.at[c].at[rows])


_sc_mesh = plsc.VectorSubcoreMesh(core_axis_name="c", subcore_axis_name="s")

_deg_call = pl.kernel(
    _deg_body,
    out_type=jax.ShapeDtypeStruct((NCORE, NP), _f32),
    mesh=_sc_mesh,
    scratch_types=[
        pltpu.VMEM((CPW, CHUNK), jnp.int32),
        pltpu.VMEM((CHUNK,), _f32),
        pltpu.SemaphoreType.DMA,
        pltpu.VMEM_SHARED((NP,), _f32),
    ],
)

_mp_call = pl.kernel(
    _mp_body,
    out_type=jax.ShapeDtypeStruct((NCORE, NP, D), _f32),
    mesh=_sc_mesh,
    scratch_types=[
        pltpu.VMEM((CPW, CHUNK), jnp.int32),
        pltpu.VMEM((2, WV, CHUNK), jnp.int32),
        pltpu.VMEM((2, CHUNK, D), _f32),
        pltpu.SemaphoreType.DMA((2,)),
        pltpu.SemaphoreType.DMA((2,)),
        pltpu.SemaphoreType.DMA,
        pltpu.VMEM_SHARED((NP, D), _f32),
    ],
)


_BLK = 1024
_GRID = NP // _BLK


def _tc1_body(x_ref, w_ref, d0_ref, d1_ref, m_ref, y_ref, dinv_ref):
    deg = d0_ref[...] + d1_ref[...] + 1.0
    dinv = lax.rsqrt(deg) * m_ref[...]
    dinv_ref[...] = dinv
    xw = jnp.dot(x_ref[...], w_ref[...], preferred_element_type=_f32,
                 precision=lax.Precision.HIGHEST)
    y_ref[...] = xw * dinv[:, None]


def _tc2_body(a0_ref, a1_ref, y1_ref, dinv_ref, b_ref, w_ref, y_ref):
    dinv = dinv_ref[...]
    acc = a0_ref[...] + a1_ref[...] - y1_ref[...]
    h = jnp.maximum(dinv[:, None] * acc + b_ref[...][None, :], 0.0)
    y_ref[...] = jnp.dot(h, w_ref[...], preferred_element_type=_f32,
                         precision=lax.Precision.HIGHEST) * dinv[:, None]


def _tc3_body(a0_ref, a1_ref, y2_ref, dinv_ref, b_ref, batch_ref, w1_ref, b1_ref,
              w2_ref, b2_ref, out_ref):
    dinv = dinv_ref[...]
    acc = a0_ref[...] + a1_ref[...] - y2_ref[...]
    h = jnp.maximum(dinv[:, None] * acc + b_ref[...][None, :], 0.0)
    gid = lax.broadcasted_iota(jnp.int32, (N_GRAPHS, NP), 0)
    onehot = (gid == batch_ref[...][None, :]).astype(_f32)
    sums = jnp.dot(onehot, h, preferred_element_type=_f32,
                   precision=lax.Precision.HIGHEST)
    counts = jnp.sum(onehot, axis=1)
    g = sums / jnp.maximum(counts, 1.0)[:, None]
    g = jnp.maximum(jnp.dot(g, w1_ref[...], preferred_element_type=_f32,
                            precision=lax.Precision.HIGHEST) + b1_ref[...][None, :], 0.0)
    out_ref[...] = jnp.dot(g, w2_ref[...], preferred_element_type=_f32,
                           precision=lax.Precision.HIGHEST) + b2_ref[...][None, :]


_row_spec = pl.BlockSpec((_BLK, D), lambda i: (i, 0))
_vec_spec = pl.BlockSpec((_BLK,), lambda i: (i,))
_full_mat = pl.BlockSpec((D, D), lambda i: (0, 0))
_full_vec = pl.BlockSpec((D,), lambda i: (0,))

_tc1_call = pl.pallas_call(
    _tc1_body,
    grid=(_GRID,),
    in_specs=[_row_spec, _full_mat, _vec_spec, _vec_spec, _vec_spec],
    out_specs=(_row_spec, _vec_spec),
    out_shape=(jax.ShapeDtypeStruct((NP, D), _f32),
               jax.ShapeDtypeStruct((NP,), _f32)),
)

_tc2_call = pl.pallas_call(
    _tc2_body,
    grid=(_GRID,),
    in_specs=[_row_spec, _row_spec, _row_spec, _vec_spec, _full_vec, _full_mat],
    out_specs=_row_spec,
    out_shape=jax.ShapeDtypeStruct((NP, D), _f32),
)

_tc3_call = pl.pallas_call(
    _tc3_body,
    out_shape=jax.ShapeDtypeStruct((N_GRAPHS, 10), _f32),
)


def kernel(x, edge_index, batch, W1, b1, W2, b2, fc1_W, fc1_b, fc2_W, fc2_b):
    src = edge_index[0].astype(jnp.int32)
    dst = edge_index[1].astype(jnp.int32)
    pad_e = E_PAD - src.shape[0]
    pad_iota = jnp.arange(pad_e, dtype=jnp.int32)
    src_r = jnp.concatenate([src, jnp.full((pad_e,), N_NODES, jnp.int32)]
                            ).reshape(NW * CPW, CHUNK)
    dst_r = jnp.concatenate([dst, pad_iota % N_NODES]).reshape(NW * CPW, CHUNK)
    dst_deg_r = jnp.concatenate(
        [dst, N_NODES + pad_iota % (NP - N_NODES)]).reshape(NW * CPW, CHUNK)

    x_p = jnp.pad(x, ((0, NP - N_NODES), (0, 0)))
    batch_p = jnp.pad(batch.astype(jnp.int32), (0, NP - N_NODES),
                      constant_values=N_GRAPHS)
    mask = (jnp.arange(NP, dtype=jnp.int32) < N_NODES).astype(_f32)
    z1d = jnp.zeros((NP,), _f32)

    deg = _deg_call(dst_deg_r, z1d)
    y1, dinv = _tc1_call(x_p, W1, deg[0], deg[1], mask)
    a = _mp_call(y1, src_r, dst_r)
    y2 = _tc2_call(a[0], a[1], y1, dinv, b1, W2)
    a = _mp_call(y2, src_r, dst_r)
    return _tc3_call(a[0], a[1], y2, dinv, b2, batch_p, fc1_W, fc1_b, fc2_W, fc2_b)
